# Initial kernel scaffold; baseline (speedup 1.0000x reference)
import jax
import jax.numpy as jnp
from jax.experimental import pallas as pl


def kernel(X, edge_index, edge_weight, W_z, b_z, W_r, b_r, W_h, b_h):
    raise NotImplementedError("write your pallas kernel here")


# speedup vs baseline: 4.7118x; 4.7118x over previous
import functools

import jax
import jax.numpy as jnp
from jax import lax
from jax.experimental import pallas as pl
from jax.experimental.pallas import tpu as pltpu
from jax.experimental.pallas import tpu_sc as plsc

N_NODES = 10000
N_EDGES = 320000
CH = 128
NC = 2
NS = 16
LANES = 16
EDGES_PER_TILE = N_EDGES // NS
CHUNK = 80
NCHUNKS = EDGES_PER_TILE // CHUNK
ROWS_A = 624
ROWS_B = 640
ZROWS = 16
DEG_PAD = 10240


def _zero_vec(ref, n):
    z = jnp.zeros((LANES,), jnp.float32)

    def body(i, _):
        ref[pl.ds(i * LANES, LANES)] = z
        return 0

    lax.fori_loop(0, n // LANES, body, 0)


def _scale_rows(rows_ref, w_ref, n_rows):
    def body(g, _):
        wv = 1.0 / w_ref[pl.ds(g * LANES, LANES)]
        for l in range(LANES):
            wl = wv[l]
            r = g * LANES + l
            for j in range(CH // LANES):
                sl = pl.ds(j * LANES, LANES)
                rows_ref[r, sl] = rows_ref[r, sl] * wl
        return 0

    lax.fori_loop(0, n_rows // LANES, body, 0)


def _sc_prop(row_hbm, col_hbm, p_hbm, ew_hbm, x_hbm, t1o_hbm, t1i_hbm,
             acc, deg, zbuf, idx_a, idx_b, idx_c, w_buf, ew_buf, rows_buf,
             sem):
    c = lax.axis_index("c")
    s = lax.axis_index("s")

    zrow = jnp.zeros((LANES,), jnp.float32)
    for i in range(ZROWS):
        for j in range(CH // LANES):
            zbuf[i, pl.ds(j * LANES, LANES)] = zrow

    start = pl.multiple_of(s * ROWS_A, 8)

    def zacc_body(i, _):
        pltpu.sync_copy(zbuf, acc.at[pl.ds(start + i * ZROWS, ZROWS)])
        return 0

    lax.fori_loop(0, ROWS_A // ZROWS, zacc_body, 0)

    @pl.when(s == NS - 1)
    def _():
        tail = pl.multiple_of((NS - 1) * ROWS_A, 8)
        pltpu.sync_copy(zbuf, acc.at[pl.ds(tail + ROWS_A, ZROWS)])

    for k in range(5):
        pltpu.sync_copy(zbuf.at[0, pl.ds(0, CH)], deg.at[pl.ds(s * 640 + k * CH, CH)])
    plsc.subcore_barrier()

    def deg_chunk(k, didx_hbm):
        base = s * EDGES_PER_TILE + k * CHUNK
        pltpu.sync_copy(didx_hbm.at[pl.ds(base, CHUNK)], idx_a)
        pltpu.sync_copy(ew_hbm.at[pl.ds(base, CHUNK)], ew_buf)
        pltpu.sync_copy(ew_buf, deg.at[idx_a], add=True)

    @pl.when(c == 0)
    def _():
        lax.fori_loop(0, NCHUNKS, lambda k, _: (deg_chunk(k, row_hbm), 0)[1], 0)

    @pl.when(c == 1)
    def _():
        lax.fori_loop(0, NCHUNKS, lambda k, _: (deg_chunk(k, col_hbm), 0)[1], 0)

    plsc.subcore_barrier()

    def prop_chunk_out(k, _):
        base = s * EDGES_PER_TILE + k * CHUNK
        pltpu.sync_copy(row_hbm.at[pl.ds(base, CHUNK)], idx_a)
        pltpu.sync_copy(col_hbm.at[pl.ds(base, CHUNK)], idx_b)
        pltpu.sync_copy(deg.at[idx_a], w_buf)
        pltpu.async_copy(x_hbm.at[idx_a], rows_buf, sem).wait()
        _scale_rows(rows_buf, w_buf, CHUNK)
        pltpu.sync_copy(rows_buf, acc.at[idx_b], add=True)
        return 0

    def prop_chunk_in(k, _):
        base = s * EDGES_PER_TILE + k * CHUNK
        pltpu.sync_copy(p_hbm.at[pl.ds(base, CHUNK)], idx_c)
        pltpu.sync_copy(row_hbm.at[pl.ds(base, CHUNK)], idx_a)
        pltpu.sync_copy(deg.at[idx_a], w_buf)
        pltpu.async_copy(col_hbm.at[idx_c], idx_a, sem).wait()
        pltpu.async_copy(row_hbm.at[idx_c], idx_b, sem).wait()
        pltpu.async_copy(x_hbm.at[idx_a], rows_buf, sem).wait()
        _scale_rows(rows_buf, w_buf, CHUNK)
        pltpu.sync_copy(rows_buf, acc.at[idx_b], add=True)
        return 0

    @pl.when(c == 0)
    def _():
        lax.fori_loop(0, NCHUNKS, prop_chunk_out, 0)

    @pl.when(c == 1)
    def _():
        lax.fori_loop(0, NCHUNKS, prop_chunk_in, 0)

    plsc.subcore_barrier()

    r0 = pl.multiple_of(s * ROWS_A, 8)

    def copy_out(dst_hbm):
        @pl.when(s < NS - 1)
        def _():
            pltpu.sync_copy(acc.at[pl.ds(r0, ROWS_A)],
                            dst_hbm.at[pl.ds(r0, ROWS_A)])

        @pl.when(s == NS - 1)
        def _():
            t0 = (NS - 1) * ROWS_A
            pltpu.sync_copy(acc.at[pl.ds(t0, ROWS_B)],
                            dst_hbm.at[pl.ds(t0, ROWS_B)])

    @pl.when(c == 0)
    def _():
        copy_out(t1o_hbm)

    @pl.when(c == 1)
    def _():
        copy_out(t1i_hbm)


def _sc_propagate(row, col, p, ew, X):
    mesh = plsc.VectorSubcoreMesh(core_axis_name="c", subcore_axis_name="s")
    f = pl.kernel(
        _sc_prop,
        out_type=[jax.ShapeDtypeStruct((N_NODES, CH), jnp.float32),
                  jax.ShapeDtypeStruct((N_NODES, CH), jnp.float32)],
        mesh=mesh,
        scratch_types=[
            pltpu.VMEM_SHARED((N_NODES, CH), jnp.float32),
            pltpu.VMEM_SHARED((DEG_PAD,), jnp.float32),
            pltpu.VMEM((ZROWS, CH), jnp.float32),
            pltpu.VMEM((CHUNK,), jnp.int32),
            pltpu.VMEM((CHUNK,), jnp.int32),
            pltpu.VMEM((CHUNK,), jnp.int32),
            pltpu.VMEM((CHUNK,), jnp.float32),
            pltpu.VMEM((CHUNK,), jnp.float32),
            pltpu.VMEM((CHUNK, CH), jnp.float32),
            pltpu.SemaphoreType.DMA,
        ],
    )
    return f(row, col, p, ew, X)


def _tc_body(x_ref, a_ref, b_ref, wz_ref, wh_ref, bz_ref, bh_ref, out_ref):
    x = x_ref[...]
    a = a_ref[...]
    b = b_ref[...]

    def dconv(w_ref, bias_ref):
        w = w_ref[...]
        acc = jnp.dot(x, w[0], preferred_element_type=jnp.float32)
        acc += jnp.dot(a, w[1], preferred_element_type=jnp.float32)
        acc += jnp.dot(b, w[2], preferred_element_type=jnp.float32)
        return acc + bias_ref[...]

    dz = dconv(wz_ref, bz_ref)
    dh = dconv(wh_ref, bh_ref)
    out_ref[...] = (1.0 - jax.nn.sigmoid(dz)) * jnp.tanh(dh)


def _tc_combine(X, T1o, T1i, Wz, Wh, bz, bh):
    blk = 1000
    grid = (N_NODES // blk,)
    io_spec = pl.BlockSpec((blk, CH), lambda i: (i, 0))
    w_spec = pl.BlockSpec((3, CH, CH), lambda i: (0, 0, 0))
    b_spec = pl.BlockSpec((1, CH), lambda i: (0, 0))
    return pl.pallas_call(
        _tc_body,
        grid=grid,
        in_specs=[io_spec, io_spec, io_spec, w_spec, w_spec, b_spec, b_spec],
        out_specs=io_spec,
        out_shape=jax.ShapeDtypeStruct((N_NODES, CH), jnp.float32),
    )(X, T1o, T1i, Wz, Wh, bz, bh)


def kernel(X, edge_index, edge_weight, W_z, b_z, W_r, b_r, W_h, b_h):
    row = edge_index[0].astype(jnp.int32)
    col = edge_index[1].astype(jnp.int32)
    p = jnp.argsort(col * N_NODES + row).astype(jnp.int32)

    T1o, T1i = _sc_propagate(row, col, p, edge_weight, X)

    def prep(W):
        return jnp.stack([W[0, 0, :CH] + W[1, 0, :CH], W[0, 1, :CH], W[1, 1, :CH]])

    Wz = prep(W_z)
    Wh = prep(W_h)
    return _tc_combine(X, T1o, T1i, Wz, Wh,
                       b_z.reshape(1, CH), b_h.reshape(1, CH))


# speedup vs baseline: 7.1865x; 1.5252x over previous
import functools

import jax
import jax.numpy as jnp
from jax import lax
from jax.experimental import pallas as pl
from jax.experimental.pallas import tpu as pltpu
from jax.experimental.pallas import tpu_sc as plsc

N_NODES = 10000
N_EDGES = 320000
CH = 128
NS = 16
LANES = 16
CHUNK = 128
NCHUNKS = N_EDGES // CHUNK
CH_MAIN = NCHUNKS // NS
CH_EXTRA = NCHUNKS - CH_MAIN * NS
ROWS_A = 624
ROWS_B = 640
ZROWS = 16
DEG_PAD = 10240


def _scale_rows(rows_ref, w_ref):
    def body(g, _):
        wv = 1.0 / w_ref[pl.ds(g * LANES, LANES)]
        for l in range(LANES):
            wl = wv[l]
            r = g * LANES + l
            for j in range(CH // LANES):
                sl = pl.ds(j * LANES, LANES)
                rows_ref[r, sl] = rows_ref[r, sl] * wl
        return 0

    lax.fori_loop(0, CHUNK // LANES, body, 0)


def _sc_prop(row_hbm, col_hbm, p_hbm, ew_hbm, x_hbm, t1o_hbm, t1i_hbm,
             acc, deg, zbuf, idx_a, idx_b, idx_c, idx_d, w_buf, ew_buf,
             rows_buf, sem, sem2):
    c = lax.axis_index("c")
    s = lax.axis_index("s")

    zrow = jnp.zeros((LANES,), jnp.float32)
    for i in range(ZROWS):
        for q in range(CH // LANES):
            zbuf[i, pl.ds(q * LANES, LANES)] = zrow

    start = pl.multiple_of(s * ROWS_A, 8)

    def zacc_body(i, _):
        pltpu.sync_copy(zbuf, acc.at[pl.ds(start + i * ZROWS, ZROWS)])
        return 0

    lax.fori_loop(0, ROWS_A // ZROWS, zacc_body, 0)

    @pl.when(s == NS - 1)
    def _():
        tail = pl.multiple_of((NS - 1) * ROWS_A, 8)
        pltpu.sync_copy(zbuf, acc.at[pl.ds(tail + ROWS_A, ZROWS)])

    for k in range(5):
        pltpu.sync_copy(zbuf.at[0, pl.ds(0, CH)], deg.at[pl.ds(s * 640 + k * CH, CH)])
    plsc.subcore_barrier()

    def deg_chunk(cid, didx_hbm):
        base = pl.multiple_of(cid * CHUNK, 8)
        cp1 = pltpu.async_copy(didx_hbm.at[pl.ds(base, CHUNK)], idx_a, sem)
        cp2 = pltpu.async_copy(ew_hbm.at[pl.ds(base, CHUNK)], ew_buf, sem2)
        cp1.wait()
        cp2.wait()
        pltpu.sync_copy(ew_buf, deg.at[idx_a], add=True)

    def deg_sweep(didx_hbm):
        def body(k, _):
            deg_chunk(s + k * NS, didx_hbm)
            return 0

        lax.fori_loop(0, CH_MAIN, body, 0)

        @pl.when(s < CH_EXTRA)
        def _():
            deg_chunk(CH_MAIN * NS + s, didx_hbm)

    @pl.when(c == 0)
    def _():
        deg_sweep(row_hbm)

    @pl.when(c == 1)
    def _():
        deg_sweep(col_hbm)

    plsc.subcore_barrier()

    def prop_chunk_out(cid):
        base = pl.multiple_of(cid * CHUNK, 8)
        cp1 = pltpu.async_copy(row_hbm.at[pl.ds(base, CHUNK)], idx_a, sem)
        cp2 = pltpu.async_copy(col_hbm.at[pl.ds(base, CHUNK)], idx_b, sem2)
        cp1.wait()
        cp2.wait()
        cp1 = pltpu.async_copy(deg.at[idx_a], w_buf, sem2)
        cp2 = pltpu.async_copy(x_hbm.at[idx_a], rows_buf, sem)
        cp1.wait()
        cp2.wait()
        _scale_rows(rows_buf, w_buf)
        pltpu.sync_copy(rows_buf, acc.at[idx_b], add=True)

    def prop_chunk_in(cid):
        base = pl.multiple_of(cid * CHUNK, 8)
        cp1 = pltpu.async_copy(p_hbm.at[pl.ds(base, CHUNK)], idx_c, sem)
        cp2 = pltpu.async_copy(row_hbm.at[pl.ds(base, CHUNK)], idx_a, sem2)
        cp1.wait()
        cp2.wait()
        cp1 = pltpu.async_copy(deg.at[idx_a], w_buf, sem2)
        cp2 = pltpu.async_copy(col_hbm.at[idx_c], idx_b, sem)
        cp1.wait()
        cp2.wait()
        cp1 = pltpu.async_copy(row_hbm.at[idx_c], idx_d, sem2)
        cp2 = pltpu.async_copy(x_hbm.at[idx_b], rows_buf, sem)
        cp1.wait()
        cp2.wait()
        _scale_rows(rows_buf, w_buf)
        pltpu.sync_copy(rows_buf, acc.at[idx_d], add=True)

    def prop_sweep(chunk_fn):
        def body(k, _):
            chunk_fn(s + k * NS)
            return 0

        lax.fori_loop(0, CH_MAIN, body, 0)

        @pl.when(s < CH_EXTRA)
        def _():
            chunk_fn(CH_MAIN * NS + s)

    @pl.when(c == 0)
    def _():
        prop_sweep(prop_chunk_out)

    @pl.when(c == 1)
    def _():
        prop_sweep(prop_chunk_in)

    plsc.subcore_barrier()

    r0 = pl.multiple_of(s * ROWS_A, 8)

    def copy_out(dst_hbm):
        @pl.when(s < NS - 1)
        def _():
            pltpu.sync_copy(acc.at[pl.ds(r0, ROWS_A)],
                            dst_hbm.at[pl.ds(r0, ROWS_A)])

        @pl.when(s == NS - 1)
        def _():
            t0 = (NS - 1) * ROWS_A
            pltpu.sync_copy(acc.at[pl.ds(t0, ROWS_B)],
                            dst_hbm.at[pl.ds(t0, ROWS_B)])

    @pl.when(c == 0)
    def _():
        copy_out(t1o_hbm)

    @pl.when(c == 1)
    def _():
        copy_out(t1i_hbm)


def _sc_propagate(row, col, p, ew, X):
    mesh = plsc.VectorSubcoreMesh(core_axis_name="c", subcore_axis_name="s")
    f = pl.kernel(
        _sc_prop,
        out_type=[jax.ShapeDtypeStruct((N_NODES, CH), jnp.float32),
                  jax.ShapeDtypeStruct((N_NODES, CH), jnp.float32)],
        mesh=mesh,
        scratch_types=[
            pltpu.VMEM_SHARED((N_NODES, CH), jnp.float32),
            pltpu.VMEM_SHARED((DEG_PAD,), jnp.float32),
            pltpu.VMEM((ZROWS, CH), jnp.float32),
            pltpu.VMEM((CHUNK,), jnp.int32),
            pltpu.VMEM((CHUNK,), jnp.int32),
            pltpu.VMEM((CHUNK,), jnp.int32),
            pltpu.VMEM((CHUNK,), jnp.int32),
            pltpu.VMEM((CHUNK,), jnp.float32),
            pltpu.VMEM((CHUNK,), jnp.float32),
            pltpu.VMEM((CHUNK, CH), jnp.float32),
            pltpu.SemaphoreType.DMA,
            pltpu.SemaphoreType.DMA,
        ],
    )
    return f(row, col, p, ew, X)


def _tc_body(x_ref, a_ref, b_ref, wz_ref, wh_ref, bz_ref, bh_ref, out_ref):
    x = x_ref[...]
    a = a_ref[...]
    b = b_ref[...]

    def dconv(w_ref, bias_ref):
        w = w_ref[...]
        acc = jnp.dot(x, w[0], preferred_element_type=jnp.float32)
        acc += jnp.dot(a, w[1], preferred_element_type=jnp.float32)
        acc += jnp.dot(b, w[2], preferred_element_type=jnp.float32)
        return acc + bias_ref[...]

    dz = dconv(wz_ref, bz_ref)
    dh = dconv(wh_ref, bh_ref)
    out_ref[...] = (1.0 - jax.nn.sigmoid(dz)) * jnp.tanh(dh)


def _tc_combine(X, T1o, T1i, Wz, Wh, bz, bh):
    blk = 1000
    grid = (N_NODES // blk,)
    io_spec = pl.BlockSpec((blk, CH), lambda i: (i, 0))
    w_spec = pl.BlockSpec((3, CH, CH), lambda i: (0, 0, 0))
    b_spec = pl.BlockSpec((1, CH), lambda i: (0, 0))
    return pl.pallas_call(
        _tc_body,
        grid=grid,
        in_specs=[io_spec, io_spec, io_spec, w_spec, w_spec, b_spec, b_spec],
        out_specs=io_spec,
        out_shape=jax.ShapeDtypeStruct((N_NODES, CH), jnp.float32),
    )(X, T1o, T1i, Wz, Wh, bz, bh)


def kernel(X, edge_index, edge_weight, W_z, b_z, W_r, b_r, W_h, b_h):
    row = edge_index[0].astype(jnp.int32)
    col = edge_index[1].astype(jnp.int32)
    p = jnp.argsort(col * N_NODES + row).astype(jnp.int32)

    T1o, T1i = _sc_propagate(row, col, p, edge_weight, X)

    def prep(W):
        return jnp.stack([W[0, 0, :CH] + W[1, 0, :CH], W[0, 1, :CH], W[1, 1, :CH]])

    Wz = prep(W_z)
    Wh = prep(W_h)
    return _tc_combine(X, T1o, T1i, Wz, Wh,
                       b_z.reshape(1, CH), b_h.reshape(1, CH))


# speedup vs baseline: 7.1911x; 1.0006x over previous
import functools

import jax
import jax.numpy as jnp
from jax import lax
from jax.experimental import pallas as pl
from jax.experimental.pallas import tpu as pltpu
from jax.experimental.pallas import tpu_sc as plsc

N_NODES = 10000
N_EDGES = 320000
CH = 128
NS = 16
LANES = 16
CHUNK = 128
NCHUNKS = N_EDGES // CHUNK
CH_MAIN = NCHUNKS // NS
CH_EXTRA = NCHUNKS - CH_MAIN * NS
ROWS_A = 624
ROWS_B = 640
ZROWS = 16
DEG_PAD = 10240


def _scale_rows(rows_ref, w_ref):
    def body(g, _):
        wv = 1.0 / w_ref[pl.ds(g * LANES, LANES)]
        for l in range(LANES):
            wl = wv[l]
            r = g * LANES + l
            for j in range(CH // LANES):
                sl = pl.ds(j * LANES, LANES)
                rows_ref[r, sl] = rows_ref[r, sl] * wl
        return 0

    lax.fori_loop(0, CHUNK // LANES, body, 0)


def _sc_prop(row_hbm, col_hbm, p_hbm, ew_hbm, x_hbm, t1o_hbm, t1i_hbm,
             acc, deg, zbuf, idx_a, idx_b, idx_c, idx_d, idx_e, w_buf, ew_buf,
             rows_buf, sem, sem2, sem3):
    c = lax.axis_index("c")
    s = lax.axis_index("s")

    zrow = jnp.zeros((LANES,), jnp.float32)
    for i in range(ZROWS):
        for q in range(CH // LANES):
            zbuf[i, pl.ds(q * LANES, LANES)] = zrow

    start = pl.multiple_of(s * ROWS_A, 8)

    def zacc_body(i, _):
        pltpu.sync_copy(zbuf, acc.at[pl.ds(start + i * ZROWS, ZROWS)])
        return 0

    lax.fori_loop(0, ROWS_A // ZROWS, zacc_body, 0)

    @pl.when(s == NS - 1)
    def _():
        tail = pl.multiple_of((NS - 1) * ROWS_A, 8)
        pltpu.sync_copy(zbuf, acc.at[pl.ds(tail + ROWS_A, ZROWS)])

    for k in range(5):
        pltpu.sync_copy(zbuf.at[0, pl.ds(0, CH)], deg.at[pl.ds(s * 640 + k * CH, CH)])
    plsc.subcore_barrier()

    def deg_chunk(cid, didx_hbm):
        base = pl.multiple_of(cid * CHUNK, 8)
        cp1 = pltpu.async_copy(didx_hbm.at[pl.ds(base, CHUNK)], idx_a, sem)
        cp2 = pltpu.async_copy(ew_hbm.at[pl.ds(base, CHUNK)], ew_buf, sem2)
        cp1.wait()
        cp2.wait()
        pltpu.sync_copy(ew_buf, deg.at[idx_a], add=True)

    def deg_sweep(didx_hbm):
        def body(k, _):
            deg_chunk(s + k * NS, didx_hbm)
            return 0

        lax.fori_loop(0, CH_MAIN, body, 0)

        @pl.when(s < CH_EXTRA)
        def _():
            deg_chunk(CH_MAIN * NS + s, didx_hbm)

    @pl.when(c == 0)
    def _():
        deg_sweep(row_hbm)

    @pl.when(c == 1)
    def _():
        deg_sweep(col_hbm)

    plsc.subcore_barrier()

    def prop_chunk_out(cid):
        base = pl.multiple_of(cid * CHUNK, 8)
        cp1 = pltpu.async_copy(row_hbm.at[pl.ds(base, CHUNK)], idx_a, sem)
        cp2 = pltpu.async_copy(col_hbm.at[pl.ds(base, CHUNK)], idx_b, sem2)
        cp1.wait()
        cp2.wait()
        cp1 = pltpu.async_copy(deg.at[idx_a], w_buf, sem2)
        cp2 = pltpu.async_copy(x_hbm.at[idx_a], rows_buf, sem)
        cp1.wait()
        cp2.wait()
        _scale_rows(rows_buf, w_buf)
        pltpu.sync_copy(rows_buf, acc.at[idx_b], add=True)

    def prop_chunk_in(cid):
        base = pl.multiple_of(cid * CHUNK, 8)
        o = pl.ds(base, CHUNK)
        cp1 = pltpu.async_copy(p_hbm.at[o], idx_c, sem)
        cp2 = pltpu.async_copy(row_hbm.at[o], idx_a, sem2)
        cp1.wait()
        cp2.wait()
        cp1 = pltpu.async_copy(col_hbm.at[idx_c], idx_b, sem)
        cp2 = pltpu.async_copy(p_hbm.at[o], idx_e, sem2)
        cp1.wait()
        cp2.wait()
        cp1 = pltpu.async_copy(x_hbm.at[idx_b], rows_buf, sem)
        cp2 = pltpu.async_copy(row_hbm.at[idx_e], idx_d, sem2)
        cp3 = pltpu.async_copy(deg.at[idx_a], w_buf, sem3)
        cp1.wait()
        cp2.wait()
        cp3.wait()
        _scale_rows(rows_buf, w_buf)
        pltpu.sync_copy(rows_buf, acc.at[idx_d], add=True)

    def prop_sweep(chunk_fn):
        def body(k, _):
            chunk_fn(s + k * NS)
            return 0

        lax.fori_loop(0, CH_MAIN, body, 0)

        @pl.when(s < CH_EXTRA)
        def _():
            chunk_fn(CH_MAIN * NS + s)

    @pl.when(c == 0)
    def _():
        prop_sweep(prop_chunk_out)

    @pl.when(c == 1)
    def _():
        prop_sweep(prop_chunk_in)

    plsc.subcore_barrier()

    r0 = pl.multiple_of(s * ROWS_A, 8)

    def copy_out(dst_hbm):
        @pl.when(s < NS - 1)
        def _():
            pltpu.sync_copy(acc.at[pl.ds(r0, ROWS_A)],
                            dst_hbm.at[pl.ds(r0, ROWS_A)])

        @pl.when(s == NS - 1)
        def _():
            t0 = (NS - 1) * ROWS_A
            pltpu.sync_copy(acc.at[pl.ds(t0, ROWS_B)],
                            dst_hbm.at[pl.ds(t0, ROWS_B)])

    @pl.when(c == 0)
    def _():
        copy_out(t1o_hbm)

    @pl.when(c == 1)
    def _():
        copy_out(t1i_hbm)


def _sc_propagate(row, col, p, ew, X):
    mesh = plsc.VectorSubcoreMesh(core_axis_name="c", subcore_axis_name="s")
    f = pl.kernel(
        _sc_prop,
        out_type=[jax.ShapeDtypeStruct((N_NODES, CH), jnp.float32),
                  jax.ShapeDtypeStruct((N_NODES, CH), jnp.float32)],
        mesh=mesh,
        scratch_types=[
            pltpu.VMEM_SHARED((N_NODES, CH), jnp.float32),
            pltpu.VMEM_SHARED((DEG_PAD,), jnp.float32),
            pltpu.VMEM((ZROWS, CH), jnp.float32),
            pltpu.VMEM((CHUNK,), jnp.int32),
            pltpu.VMEM((CHUNK,), jnp.int32),
            pltpu.VMEM((CHUNK,), jnp.int32),
            pltpu.VMEM((CHUNK,), jnp.int32),
            pltpu.VMEM((CHUNK,), jnp.int32),
            pltpu.VMEM((CHUNK,), jnp.float32),
            pltpu.VMEM((CHUNK,), jnp.float32),
            pltpu.VMEM((CHUNK, CH), jnp.float32),
            pltpu.SemaphoreType.DMA,
            pltpu.SemaphoreType.DMA,
            pltpu.SemaphoreType.DMA,
        ],
    )
    return f(row, col, p, ew, X)


def _tc_body(x_ref, a_ref, b_ref, wz_ref, wh_ref, bz_ref, bh_ref, out_ref):
    x = x_ref[...]
    a = a_ref[...]
    b = b_ref[...]

    def dconv(w_ref, bias_ref):
        w = w_ref[...]
        acc = jnp.dot(x, w[0], preferred_element_type=jnp.float32)
        acc += jnp.dot(a, w[1], preferred_element_type=jnp.float32)
        acc += jnp.dot(b, w[2], preferred_element_type=jnp.float32)
        return acc + bias_ref[...]

    dz = dconv(wz_ref, bz_ref)
    dh = dconv(wh_ref, bh_ref)
    out_ref[...] = (1.0 - jax.nn.sigmoid(dz)) * jnp.tanh(dh)


def _tc_combine(X, T1o, T1i, Wz, Wh, bz, bh):
    blk = 1000
    grid = (N_NODES // blk,)
    io_spec = pl.BlockSpec((blk, CH), lambda i: (i, 0))
    w_spec = pl.BlockSpec((3, CH, CH), lambda i: (0, 0, 0))
    b_spec = pl.BlockSpec((1, CH), lambda i: (0, 0))
    return pl.pallas_call(
        _tc_body,
        grid=grid,
        in_specs=[io_spec, io_spec, io_spec, w_spec, w_spec, b_spec, b_spec],
        out_specs=io_spec,
        out_shape=jax.ShapeDtypeStruct((N_NODES, CH), jnp.float32),
    )(X, T1o, T1i, Wz, Wh, bz, bh)


def kernel(X, edge_index, edge_weight, W_z, b_z, W_r, b_r, W_h, b_h):
    row = edge_index[0].astype(jnp.int32)
    col = edge_index[1].astype(jnp.int32)
    p = jnp.argsort(col * N_NODES + row).astype(jnp.int32)

    T1o, T1i = _sc_propagate(row, col, p, edge_weight, X)

    def prep(W):
        return jnp.stack([W[0, 0, :CH] + W[1, 0, :CH], W[0, 1, :CH], W[1, 1, :CH]])

    Wz = prep(W_z)
    Wh = prep(W_h)
    return _tc_combine(X, T1o, T1i, Wz, Wh,
                       b_z.reshape(1, CH), b_h.reshape(1, CH))


# speedup vs baseline: 9.7345x; 1.3537x over previous
import functools

import jax
import jax.numpy as jnp
from jax import lax
from jax.experimental import pallas as pl
from jax.experimental.pallas import tpu as pltpu
from jax.experimental.pallas import tpu_sc as plsc

N_NODES = 10000
N_EDGES = 320000
CH = 128
NS = 16
LANES = 16
CHUNK = 128
NCHUNKS = N_EDGES // CHUNK
CH_MAIN = NCHUNKS // NS
CH_EXTRA = NCHUNKS - CH_MAIN * NS
ROWS_A = 624
ROWS_B = 640
ZROWS = 16
DEG_PAD = 10240


def _scale_rows(rows_ref, w_ref):
    def body(g, _):
        wv = 1.0 / w_ref[pl.ds(g * LANES, LANES)]
        for l in range(LANES):
            wl = wv[l]
            r = g * LANES + l
            for j in range(CH // LANES):
                sl = pl.ds(j * LANES, LANES)
                rows_ref[r, sl] = rows_ref[r, sl] * wl
        return 0

    lax.fori_loop(0, CHUNK // LANES, body, 0)


def _sc_prop(row_hbm, col_hbm, p_hbm, ew_hbm, x_hbm, t1o_hbm, t1i_hbm,
             acc, deg, zbuf, idx_a, idx_b, idx_c, idx_d, idx_e, idx_f,
             w_buf, ew_buf, rows_buf, sem, sem2, sem3, sem4):
    c = lax.axis_index("c")
    s = lax.axis_index("s")

    zrow = jnp.zeros((LANES,), jnp.float32)
    for i in range(ZROWS):
        for q in range(CH // LANES):
            zbuf[i, pl.ds(q * LANES, LANES)] = zrow

    start = pl.multiple_of(s * ROWS_A, 8)

    def zacc_body(i, _):
        pltpu.sync_copy(zbuf, acc.at[pl.ds(start + i * ZROWS, ZROWS)])
        return 0

    lax.fori_loop(0, ROWS_A // ZROWS, zacc_body, 0)

    @pl.when(s == NS - 1)
    def _():
        tail = pl.multiple_of((NS - 1) * ROWS_A, 8)
        pltpu.sync_copy(zbuf, acc.at[pl.ds(tail + ROWS_A, ZROWS)])

    for k in range(5):
        pltpu.sync_copy(zbuf.at[0, pl.ds(0, CH)], deg.at[pl.ds(s * 640 + k * CH, CH)])
    plsc.subcore_barrier()

    def deg_chunk(cid, didx_hbm):
        base = pl.multiple_of(cid * CHUNK, 8)
        cp1 = pltpu.async_copy(didx_hbm.at[pl.ds(base, CHUNK)], idx_a, sem)
        cp2 = pltpu.async_copy(ew_hbm.at[pl.ds(base, CHUNK)], ew_buf, sem2)
        cp1.wait()
        cp2.wait()
        pltpu.sync_copy(ew_buf, deg.at[idx_a], add=True)

    def deg_sweep(didx_hbm):
        def body(k, _):
            deg_chunk(s + k * NS, didx_hbm)
            return 0

        lax.fori_loop(0, CH_MAIN, body, 0)

        @pl.when(s < CH_EXTRA)
        def _():
            deg_chunk(CH_MAIN * NS + s, didx_hbm)

    @pl.when(c == 0)
    def _():
        deg_sweep(row_hbm)

    @pl.when(c == 1)
    def _():
        deg_sweep(col_hbm)

    plsc.subcore_barrier()

    def prop_chunk_out(cid):
        base = pl.multiple_of(cid * CHUNK, 8)
        cp1 = pltpu.async_copy(row_hbm.at[pl.ds(base, CHUNK)], idx_a, sem)
        cp2 = pltpu.async_copy(col_hbm.at[pl.ds(base, CHUNK)], idx_b, sem2)
        cp1.wait()
        cp2.wait()
        cp1 = pltpu.async_copy(deg.at[idx_a], w_buf, sem2)
        cp2 = pltpu.async_copy(x_hbm.at[idx_a], rows_a_half, sem)
        cp1.wait()
        cp2.wait()
        _scale_rows(rows_a_half, w_buf)
        pltpu.sync_copy(rows_a_half, acc.at[idx_b], add=True)

    rows2 = (rows_buf.at[pl.ds(0, CHUNK)], rows_buf.at[pl.ds(CHUNK, CHUNK)])
    rows_a_half = rows2[0]
    wb2 = (w_buf, ew_buf)
    dst2 = (idx_d, idx_f)
    semx = (sem3, sem4)

    def in_stage12(cid):
        base = pl.multiple_of(cid * CHUNK, 8)
        o = pl.ds(base, CHUNK)
        cp1 = pltpu.async_copy(p_hbm.at[o], idx_c, sem)
        cp2 = pltpu.async_copy(row_hbm.at[o], idx_a, sem2)
        cp1.wait()
        cp2.wait()
        cp1 = pltpu.async_copy(col_hbm.at[idx_c], idx_b, sem)
        cp2 = pltpu.async_copy(p_hbm.at[o], idx_e, sem2)
        cp1.wait()
        cp2.wait()

    def in_fire3(b):
        cpx = pltpu.async_copy(x_hbm.at[idx_b], rows2[b], semx[b])
        cp2 = pltpu.async_copy(row_hbm.at[idx_e], dst2[b], sem)
        cp3 = pltpu.async_copy(deg.at[idx_a], wb2[b], sem2)
        return cpx, cp2, cp3

    def in_process(b):
        def body(g, _):
            wv = 1.0 / wb2[b][pl.ds(g * LANES, LANES)]
            for l in range(LANES):
                wl = wv[l]
                r = g * LANES + l
                for q in range(CH // LANES):
                    sl = pl.ds(q * LANES, LANES)
                    rows2[b][r, sl] = rows2[b][r, sl] * wl
            return 0

        lax.fori_loop(0, CHUNK // LANES, body, 0)
        pltpu.sync_copy(rows2[b], acc.at[dst2[b]], add=True)

    def in_wait_x(b):
        pltpu.make_async_copy(x_hbm.at[idx_b], rows2[b], semx[b]).wait()

    def in_sub(k, b, prefetch_cid):
        if prefetch_cid is not None:
            in_stage12(prefetch_cid)
            cpx, cp2, cp3 = in_fire3(1 - b)
            in_wait_x(b)
            in_process(b)
            cp2.wait()
            cp3.wait()
        else:
            in_wait_x(b)
            in_process(b)

    def in_sweep():
        @pl.when(s < CH_EXTRA)
        def _():
            in_stage12(CH_MAIN * NS + s)
            cpx, cp2, cp3 = in_fire3(0)
            cpx.wait()
            cp2.wait()
            cp3.wait()
            in_process(0)

        in_stage12(s)
        cpx, cp2, cp3 = in_fire3(0)
        cp2.wait()
        cp3.wait()

        def body(g, _):
            k0 = 2 * g
            in_sub(k0, 0, s + (k0 + 1) * NS)
            in_sub(k0 + 1, 1, s + (k0 + 2) * NS)
            return 0

        lax.fori_loop(0, (CH_MAIN - 2) // 2, body, 0)
        in_sub(CH_MAIN - 2, 0, s + (CH_MAIN - 1) * NS)
        in_sub(CH_MAIN - 1, 1, None)

    def prop_sweep(chunk_fn):
        def body(k, _):
            chunk_fn(s + k * NS)
            return 0

        lax.fori_loop(0, CH_MAIN, body, 0)

        @pl.when(s < CH_EXTRA)
        def _():
            chunk_fn(CH_MAIN * NS + s)

    @pl.when(c == 0)
    def _():
        prop_sweep(prop_chunk_out)

    @pl.when(c == 1)
    def _():
        in_sweep()

    plsc.subcore_barrier()

    r0 = pl.multiple_of(s * ROWS_A, 8)

    def copy_out(dst_hbm):
        @pl.when(s < NS - 1)
        def _():
            pltpu.sync_copy(acc.at[pl.ds(r0, ROWS_A)],
                            dst_hbm.at[pl.ds(r0, ROWS_A)])

        @pl.when(s == NS - 1)
        def _():
            t0 = (NS - 1) * ROWS_A
            pltpu.sync_copy(acc.at[pl.ds(t0, ROWS_B)],
                            dst_hbm.at[pl.ds(t0, ROWS_B)])

    @pl.when(c == 0)
    def _():
        copy_out(t1o_hbm)

    @pl.when(c == 1)
    def _():
        copy_out(t1i_hbm)


def _sc_propagate(row, col, p, ew, X):
    mesh = plsc.VectorSubcoreMesh(core_axis_name="c", subcore_axis_name="s")
    f = pl.kernel(
        _sc_prop,
        out_type=[jax.ShapeDtypeStruct((N_NODES, CH), jnp.float32),
                  jax.ShapeDtypeStruct((N_NODES, CH), jnp.float32)],
        mesh=mesh,
        scratch_types=[
            pltpu.VMEM_SHARED((N_NODES, CH), jnp.float32),
            pltpu.VMEM_SHARED((DEG_PAD,), jnp.float32),
            pltpu.VMEM((ZROWS, CH), jnp.float32),
            pltpu.VMEM((CHUNK,), jnp.int32),
            pltpu.VMEM((CHUNK,), jnp.int32),
            pltpu.VMEM((CHUNK,), jnp.int32),
            pltpu.VMEM((CHUNK,), jnp.int32),
            pltpu.VMEM((CHUNK,), jnp.int32),
            pltpu.VMEM((CHUNK,), jnp.int32),
            pltpu.VMEM((CHUNK,), jnp.float32),
            pltpu.VMEM((CHUNK,), jnp.float32),
            pltpu.VMEM((2 * CHUNK, CH), jnp.float32),
            pltpu.SemaphoreType.DMA,
            pltpu.SemaphoreType.DMA,
            pltpu.SemaphoreType.DMA,
            pltpu.SemaphoreType.DMA,
        ],
    )
    return f(row, col, p, ew, X)


def _tc_body(x_ref, a_ref, b_ref, wz_ref, wh_ref, bz_ref, bh_ref, out_ref):
    x = x_ref[...]
    a = a_ref[...]
    b = b_ref[...]

    def dconv(w_ref, bias_ref):
        w = w_ref[...]
        acc = jnp.dot(x, w[0], preferred_element_type=jnp.float32)
        acc += jnp.dot(a, w[1], preferred_element_type=jnp.float32)
        acc += jnp.dot(b, w[2], preferred_element_type=jnp.float32)
        return acc + bias_ref[...]

    dz = dconv(wz_ref, bz_ref)
    dh = dconv(wh_ref, bh_ref)
    out_ref[...] = (1.0 - jax.nn.sigmoid(dz)) * jnp.tanh(dh)


def _tc_combine(X, T1o, T1i, Wz, Wh, bz, bh):
    blk = 1000
    grid = (N_NODES // blk,)
    io_spec = pl.BlockSpec((blk, CH), lambda i: (i, 0))
    w_spec = pl.BlockSpec((3, CH, CH), lambda i: (0, 0, 0))
    b_spec = pl.BlockSpec((1, CH), lambda i: (0, 0))
    return pl.pallas_call(
        _tc_body,
        grid=grid,
        in_specs=[io_spec, io_spec, io_spec, w_spec, w_spec, b_spec, b_spec],
        out_specs=io_spec,
        out_shape=jax.ShapeDtypeStruct((N_NODES, CH), jnp.float32),
    )(X, T1o, T1i, Wz, Wh, bz, bh)


def kernel(X, edge_index, edge_weight, W_z, b_z, W_r, b_r, W_h, b_h):
    row = edge_index[0].astype(jnp.int32)
    col = edge_index[1].astype(jnp.int32)
    p = jnp.argsort(col * N_NODES + row).astype(jnp.int32)

    T1o, T1i = _sc_propagate(row, col, p, edge_weight, X)

    def prep(W):
        return jnp.stack([W[0, 0, :CH] + W[1, 0, :CH], W[0, 1, :CH], W[1, 1, :CH]])

    Wz = prep(W_z)
    Wh = prep(W_h)
    return _tc_combine(X, T1o, T1i, Wz, Wh,
                       b_z.reshape(1, CH), b_h.reshape(1, CH))


# speedup vs baseline: 9.7458x; 1.0012x over previous
import jax
import jax.numpy as jnp
from jax import lax
from jax.experimental import pallas as pl
from jax.experimental.pallas import tpu as pltpu
from jax.experimental.pallas import tpu_sc as plsc

N_NODES = 10000
N_EDGES = 320000
CH = 128
NS = 16
LANES = 16
CHUNK = 128
NCHUNKS = N_EDGES // CHUNK
CH_MAIN = NCHUNKS // NS
CH_EXTRA = NCHUNKS - CH_MAIN * NS
ROWS_A = 624
ROWS_B = 640
ZROWS = 16
DEG_PAD = 10240


def _scale_rows(rows_ref, w_ref):
    def body(g, _):
        wv = 1.0 / w_ref[pl.ds(g * LANES, LANES)]
        for l in range(LANES):
            wl = wv[l]
            r = g * LANES + l
            for j in range(CH // LANES):
                sl = pl.ds(j * LANES, LANES)
                rows_ref[r, sl] = rows_ref[r, sl] * wl
        return 0

    lax.fori_loop(0, CHUNK // LANES, body, 0)


def _sc_prop(row_hbm, col_hbm, p_hbm, ew_hbm, x_hbm, t1o_hbm, t1i_hbm,
             acc, deg, zbuf, idx_a, idx_b, idx_c, idx_d, idx_e, idx_f,
             w_buf, ew_buf, rows_buf, sem, sem2, sem3, sem4):
    c = lax.axis_index("c")
    s = lax.axis_index("s")

    zrow = jnp.zeros((LANES,), jnp.float32)
    for i in range(ZROWS):
        for q in range(CH // LANES):
            zbuf[i, pl.ds(q * LANES, LANES)] = zrow

    start = pl.multiple_of(s * ROWS_A, 8)

    def zacc_body(i, _):
        pltpu.sync_copy(zbuf, acc.at[pl.ds(start + i * ZROWS, ZROWS)])
        return 0

    lax.fori_loop(0, ROWS_A // ZROWS, zacc_body, 0)

    @pl.when(s == NS - 1)
    def _():
        tail = pl.multiple_of((NS - 1) * ROWS_A, 8)
        pltpu.sync_copy(zbuf, acc.at[pl.ds(tail + ROWS_A, ZROWS)])

    for k in range(5):
        pltpu.sync_copy(zbuf.at[0, pl.ds(0, CH)], deg.at[pl.ds(s * 640 + k * CH, CH)])
    plsc.subcore_barrier()

    def deg_chunk(cid, didx_hbm):
        base = pl.multiple_of(cid * CHUNK, 8)
        cp1 = pltpu.async_copy(didx_hbm.at[pl.ds(base, CHUNK)], idx_a, sem)
        cp2 = pltpu.async_copy(ew_hbm.at[pl.ds(base, CHUNK)], ew_buf, sem2)
        cp1.wait()
        cp2.wait()
        pltpu.sync_copy(ew_buf, deg.at[idx_a], add=True)

    def deg_sweep(didx_hbm):
        def body(k, _):
            deg_chunk(s + k * NS, didx_hbm)
            return 0

        lax.fori_loop(0, CH_MAIN, body, 0)

        @pl.when(s < CH_EXTRA)
        def _():
            deg_chunk(CH_MAIN * NS + s, didx_hbm)

    @pl.when(c == 0)
    def _():
        deg_sweep(row_hbm)

    @pl.when(c == 1)
    def _():
        deg_sweep(col_hbm)

    plsc.subcore_barrier()

    def prop_chunk_out(cid):
        base = pl.multiple_of(cid * CHUNK, 8)
        cp1 = pltpu.async_copy(row_hbm.at[pl.ds(base, CHUNK)], idx_a, sem)
        cp2 = pltpu.async_copy(col_hbm.at[pl.ds(base, CHUNK)], idx_b, sem2)
        cp1.wait()
        cp2.wait()
        cp1 = pltpu.async_copy(deg.at[idx_a], w_buf, sem2)
        cp2 = pltpu.async_copy(x_hbm.at[idx_a], rows_a_half, sem)
        cp1.wait()
        cp2.wait()
        _scale_rows(rows_a_half, w_buf)
        pltpu.sync_copy(rows_a_half, acc.at[idx_b], add=True)

    rows2 = (rows_buf.at[pl.ds(0, CHUNK)], rows_buf.at[pl.ds(CHUNK, CHUNK)])
    rows_a_half = rows2[0]
    wb2 = (w_buf, ew_buf)
    dst2 = (idx_d, idx_f)
    semx = (sem3, sem4)

    def in_stage12(cid):
        base = pl.multiple_of(cid * CHUNK, 8)
        o = pl.ds(base, CHUNK)
        cp1 = pltpu.async_copy(p_hbm.at[o], idx_c, sem)
        cp2 = pltpu.async_copy(row_hbm.at[o], idx_a, sem2)
        cp1.wait()
        cp2.wait()
        cp1 = pltpu.async_copy(col_hbm.at[idx_c], idx_b, sem)
        cp2 = pltpu.async_copy(p_hbm.at[o], idx_e, sem2)
        cp1.wait()
        cp2.wait()

    def in_fire3(b):
        cpx = pltpu.async_copy(x_hbm.at[idx_b], rows2[b], semx[b])
        cp2 = pltpu.async_copy(row_hbm.at[idx_e], dst2[b], sem)
        cp3 = pltpu.async_copy(deg.at[idx_a], wb2[b], sem2)
        return cpx, cp2, cp3

    def in_process(b):
        def body(g, _):
            wv = 1.0 / wb2[b][pl.ds(g * LANES, LANES)]
            for l in range(LANES):
                wl = wv[l]
                r = g * LANES + l
                for q in range(CH // LANES):
                    sl = pl.ds(q * LANES, LANES)
                    rows2[b][r, sl] = rows2[b][r, sl] * wl
            return 0

        lax.fori_loop(0, CHUNK // LANES, body, 0)
        pltpu.sync_copy(rows2[b], acc.at[dst2[b]], add=True)

    def in_wait_x(b):
        pltpu.make_async_copy(x_hbm.at[idx_b], rows2[b], semx[b]).wait()

    def in_sub(k, b, prefetch_cid):
        if prefetch_cid is not None:
            in_stage12(prefetch_cid)
            cpx, cp2, cp3 = in_fire3(1 - b)
            in_wait_x(b)
            in_process(b)
            cp2.wait()
            cp3.wait()
        else:
            in_wait_x(b)
            in_process(b)

    def in_sweep():
        @pl.when(s < CH_EXTRA)
        def _():
            in_stage12(CH_MAIN * NS + s)
            cpx, cp2, cp3 = in_fire3(0)
            cpx.wait()
            cp2.wait()
            cp3.wait()
            in_process(0)

        in_stage12(s)
        cpx, cp2, cp3 = in_fire3(0)
        cp2.wait()
        cp3.wait()

        def body(g, _):
            k0 = 2 * g
            in_sub(k0, 0, s + (k0 + 1) * NS)
            in_sub(k0 + 1, 1, s + (k0 + 2) * NS)
            return 0

        lax.fori_loop(0, (CH_MAIN - 2) // 2, body, 0)
        in_sub(CH_MAIN - 2, 0, s + (CH_MAIN - 1) * NS)
        in_sub(CH_MAIN - 1, 1, None)

    def prop_sweep(chunk_fn):
        def body(k, _):
            chunk_fn(s + k * NS)
            return 0

        lax.fori_loop(0, CH_MAIN, body, 0)

        @pl.when(s < CH_EXTRA)
        def _():
            chunk_fn(CH_MAIN * NS + s)

    @pl.when(c == 0)
    def _():
        prop_sweep(prop_chunk_out)

    @pl.when(c == 1)
    def _():
        in_sweep()

    plsc.subcore_barrier()

    r0 = pl.multiple_of(s * ROWS_A, 8)

    def copy_out(dst_hbm):
        @pl.when(s < NS - 1)
        def _():
            pltpu.sync_copy(acc.at[pl.ds(r0, ROWS_A)],
                            dst_hbm.at[pl.ds(r0, ROWS_A)])

        @pl.when(s == NS - 1)
        def _():
            t0 = (NS - 1) * ROWS_A
            pltpu.sync_copy(acc.at[pl.ds(t0, ROWS_B)],
                            dst_hbm.at[pl.ds(t0, ROWS_B)])

    @pl.when(c == 0)
    def _():
        copy_out(t1o_hbm)

    @pl.when(c == 1)
    def _():
        copy_out(t1i_hbm)


def _sc_propagate(row, col, p, ew, X):
    mesh = plsc.VectorSubcoreMesh(core_axis_name="c", subcore_axis_name="s")
    f = pl.kernel(
        _sc_prop,
        out_type=[jax.ShapeDtypeStruct((N_NODES, CH), jnp.float32),
                  jax.ShapeDtypeStruct((N_NODES, CH), jnp.float32)],
        mesh=mesh,
        scratch_types=[
            pltpu.VMEM_SHARED((N_NODES, CH), jnp.float32),
            pltpu.VMEM_SHARED((DEG_PAD,), jnp.float32),
            pltpu.VMEM((ZROWS, CH), jnp.float32),
            pltpu.VMEM((CHUNK,), jnp.int32),
            pltpu.VMEM((CHUNK,), jnp.int32),
            pltpu.VMEM((CHUNK,), jnp.int32),
            pltpu.VMEM((CHUNK,), jnp.int32),
            pltpu.VMEM((CHUNK,), jnp.int32),
            pltpu.VMEM((CHUNK,), jnp.int32),
            pltpu.VMEM((CHUNK,), jnp.float32),
            pltpu.VMEM((CHUNK,), jnp.float32),
            pltpu.VMEM((2 * CHUNK, CH), jnp.float32),
            pltpu.SemaphoreType.DMA,
            pltpu.SemaphoreType.DMA,
            pltpu.SemaphoreType.DMA,
            pltpu.SemaphoreType.DMA,
        ],
    )
    return f(row, col, p, ew, X)


def _tc_body(x_ref, a_ref, b_ref, wz_ref, wh_ref, bz_ref, bh_ref, out_ref):
    x = x_ref[...]
    a = a_ref[...]
    b = b_ref[...]

    def dconv(w_ref, bias_ref):
        w = w_ref[...]
        acc = jnp.dot(x, w[0], preferred_element_type=jnp.float32)
        acc += jnp.dot(a, w[1], preferred_element_type=jnp.float32)
        acc += jnp.dot(b, w[2], preferred_element_type=jnp.float32)
        return acc + bias_ref[...]

    dz = dconv(wz_ref, bz_ref)
    dh = dconv(wh_ref, bh_ref)
    out_ref[...] = (1.0 - jax.nn.sigmoid(dz)) * jnp.tanh(dh)


def _tc_combine(X, T1o, T1i, Wz, Wh, bz, bh):
    blk = 1000
    grid = (N_NODES // blk,)
    io_spec = pl.BlockSpec((blk, CH), lambda i: (i, 0))
    w_spec = pl.BlockSpec((3, CH, CH), lambda i: (0, 0, 0))
    b_spec = pl.BlockSpec((1, CH), lambda i: (0, 0))
    return pl.pallas_call(
        _tc_body,
        grid=grid,
        in_specs=[io_spec, io_spec, io_spec, w_spec, w_spec, b_spec, b_spec],
        out_specs=io_spec,
        out_shape=jax.ShapeDtypeStruct((N_NODES, CH), jnp.float32),
    )(X, T1o, T1i, Wz, Wh, bz, bh)


def kernel(X, edge_index, edge_weight, W_z, b_z, W_r, b_r, W_h, b_h):
    row = edge_index[0].astype(jnp.int32)
    col = edge_index[1].astype(jnp.int32)
    p = jnp.argsort(col * N_NODES + row).astype(jnp.int32)

    T1o, T1i = _sc_propagate(row, col, p, edge_weight, X)

    def prep(W):
        return jnp.stack([W[0, 0, :CH] + W[1, 0, :CH], W[0, 1, :CH], W[1, 1, :CH]])

    Wz = prep(W_z)
    Wh = prep(W_h)
    return _tc_combine(X, T1o, T1i, Wz, Wh,
                       b_z.reshape(1, CH), b_h.reshape(1, CH))
